# baseline (speedup 1.0000x reference)
import jax
import jax.numpy as jnp
from jax import lax
from jax.experimental import pallas as pl
from jax.experimental.pallas import tpu as pltpu
from jax.experimental.pallas import tpu_sc as plsc

N_ENT = 10000
N_USERS = 2048
N_EDGES = 320000
D = 128
NREL = 11
N_HOPS = 2

NC, NS = 2, 16
NW = NC * NS
CHUNK = 128
SUB = 4
SUPER = SUB * CHUNK
SUPERS_PER_TILE = -(-N_EDGES // (SUPER * NW))
N_SUPERS = SUPERS_PER_TILE * NW
PAD_CHUNKS = (N_SUPERS + 2 * NW) * SUB
PAD_EDGES = PAD_CHUNKS * CHUNK
ROWS_PER_TILE = 640
ACC_ROWS = ROWS_PER_TILE * NS
CNT_W = 16


def _zero_rows(rows_v):
    zeros16 = jnp.zeros((16,), jnp.float32)

    def fill_zero(i, carry):
        for dcol in range(D // 16):
            rows_v[i, pl.ds(dcol * 16, 16)] = zeros16
        return carry

    lax.fori_loop(0, CHUNK, fill_zero, 0)


def _agg_body(table, gidx2d, head2d, sums_out,
              acc, gA, hA, gB, hB, zidx_v, rows0, rows1,
              sem0, sem1, isemA, isemB):
    cid = lax.axis_index("c")
    sid = lax.axis_index("s")
    wid = cid * NS + sid

    iota16 = lax.iota(jnp.int32, 16)

    def fill_zidx(base):
        for jj in range(CHUNK // 16):
            zidx_v[pl.ds(jj * 16, 16)] = base + jj * 16 + iota16

    _zero_rows(rows0)
    for j in range(ROWS_PER_TILE // CHUNK):
        base = sid * ROWS_PER_TILE + j * CHUNK
        fill_zidx(base)
        pltpu.sync_copy(rows0, acc.at[zidx_v])

    plsc.subcore_barrier()

    rows = (rows0, rows1)
    sems = (sem0, sem1)
    gbuf = (gA, gB)
    hbuf = (hA, hB)
    isems = (isemA, isemB)

    def idx_issue(super_i, slot):
        off = (wid + super_i * NW) * SUB
        pltpu.async_copy(gidx2d.at[pl.ds(off, SUB)], gbuf[slot], isems[slot])
        pltpu.async_copy(head2d.at[pl.ds(off, SUB)], hbuf[slot], isems[slot])

    def idx_wait(slot):
        pltpu.make_async_copy(gidx2d.at[pl.ds(0, SUB)], gbuf[slot],
                              isems[slot]).wait()
        pltpu.make_async_copy(head2d.at[pl.ds(0, SUB)], hbuf[slot],
                              isems[slot]).wait()

    def gather_issue(slot, j, par):
        pltpu.async_copy(table.at[gbuf[slot].at[j]], rows[par], sems[par])

    def gather_wait(slot, j, par):
        pltpu.make_async_copy(table.at[gbuf[slot].at[j]], rows[par],
                              sems[par]).wait()

    idx_issue(0, 0)
    idx_wait(0)
    idx_issue(1, 1)
    gather_issue(0, 0, 0)

    def pair_body(i, carry):
        for p in (0, 1):
            super_i = 2 * i + p
            cur, nxt = (0, 1) if p == 0 else (1, 0)
            for j in range(SUB):
                gather_wait(cur, j, j % 2)
                if j + 1 < SUB:
                    gather_issue(cur, j + 1, (j + 1) % 2)
                else:
                    idx_wait(nxt)
                    gather_issue(nxt, 0, 0)
                pltpu.sync_copy(rows[j % 2], acc.at[hbuf[cur].at[j]], add=True)
            idx_issue(super_i + 2, cur)
        return carry

    lax.fori_loop(0, SUPERS_PER_TILE // 2, pair_body, 0)

    idx_wait(1)
    gather_wait(0, 0, 0)

    plsc.subcore_barrier()

    for j in range(ROWS_PER_TILE // CHUNK):
        base = sid * ROWS_PER_TILE + j * CHUNK
        fill_zidx(base)
        pltpu.sync_copy(acc.at[zidx_v], rows0)
        pltpu.sync_copy(rows0, sums_out.at[cid, pl.ds(base, CHUNK)])


def _sc_agg(table, gidx2d, head2d):
    fn = pl.kernel(
        _agg_body,
        out_type=jax.ShapeDtypeStruct((NC, ACC_ROWS, D), jnp.float32),
        mesh=plsc.VectorSubcoreMesh(core_axis_name="c", subcore_axis_name="s",
                                    num_cores=NC, num_subcores=NS),
        scratch_types=[
            pltpu.VMEM_SHARED((ACC_ROWS, D), jnp.float32),
            pltpu.VMEM((SUB, CHUNK), jnp.int32),
            pltpu.VMEM((SUB, CHUNK), jnp.int32),
            pltpu.VMEM((SUB, CHUNK), jnp.int32),
            pltpu.VMEM((SUB, CHUNK), jnp.int32),
            pltpu.VMEM((CHUNK,), jnp.int32),
            pltpu.VMEM((CHUNK, D), jnp.float32),
            pltpu.VMEM((CHUNK, D), jnp.float32),
            pltpu.SemaphoreType.DMA,
            pltpu.SemaphoreType.DMA,
            pltpu.SemaphoreType.DMA,
            pltpu.SemaphoreType.DMA,
        ],
    )
    return fn(table, gidx2d, head2d)


def _cnt_body(head2d, cnts_out, cnt_acc, hidx_v, zidx_v, ones_v):
    cid = lax.axis_index("c")
    sid = lax.axis_index("s")
    wid = cid * NS + sid

    zeros16 = jnp.zeros((16,), jnp.float32)
    ones16 = jnp.ones((16,), jnp.float32)
    iota16 = lax.iota(jnp.int32, 16)

    def fill_zidx(base):
        for jj in range(CHUNK // 16):
            zidx_v[pl.ds(jj * 16, 16)] = base + jj * 16 + iota16

    def fill_ones(val):
        def body(i, carry):
            ones_v[i, :] = val
            return carry

        lax.fori_loop(0, CHUNK, body, 0)

    fill_ones(zeros16)
    for j in range(ROWS_PER_TILE // CHUNK):
        base = sid * ROWS_PER_TILE + j * CHUNK
        fill_zidx(base)
        pltpu.sync_copy(ones_v, cnt_acc.at[zidx_v])
    fill_ones(ones16)

    plsc.subcore_barrier()

    def super_body(i, carry):
        s = wid + i * NW

        @pl.when(s < N_SUPERS)
        def _():
            pltpu.sync_copy(head2d.at[pl.ds(s * SUB, SUB)], hidx_v)
            for j in range(SUB):
                pltpu.sync_copy(ones_v, cnt_acc.at[hidx_v.at[j]], add=True)

        return carry

    lax.fori_loop(0, SUPERS_PER_TILE, super_body, 0)

    plsc.subcore_barrier()

    for j in range(ROWS_PER_TILE // CHUNK):
        base = sid * ROWS_PER_TILE + j * CHUNK
        fill_zidx(base)
        pltpu.sync_copy(cnt_acc.at[zidx_v], ones_v)
        pltpu.sync_copy(ones_v, cnts_out.at[cid, pl.ds(base, CHUNK)])


def _sc_cnt(head2d):
    fn = pl.kernel(
        _cnt_body,
        out_type=jax.ShapeDtypeStruct((NC, ACC_ROWS, CNT_W), jnp.float32),
        mesh=plsc.VectorSubcoreMesh(core_axis_name="c", subcore_axis_name="s",
                                    num_cores=NC, num_subcores=NS),
        scratch_types=[
            pltpu.VMEM_SHARED((ACC_ROWS, CNT_W), jnp.float32),
            pltpu.VMEM((SUB, CHUNK), jnp.int32),
            pltpu.VMEM((CHUNK,), jnp.int32),
            pltpu.VMEM((CHUNK, CNT_W), jnp.float32),
        ],
    )
    return fn(head2d)


EBLK = 2048
N_TBL_BLKS = ACC_ROWS // EBLK


def _scale_tbl_body(e_ref, w_ref, out_ref):
    r = pl.program_id(0)
    out_ref[...] = e_ref[...] * w_ref[pl.ds(r, 1), :]


def _scale_tbl(e_pad, weight):
    return pl.pallas_call(
        _scale_tbl_body,
        grid=(NREL, N_TBL_BLKS),
        in_specs=[pl.BlockSpec((EBLK, D), lambda r, i: (i, 0)),
                  pl.BlockSpec((NREL, D), lambda r, i: (0, 0))],
        out_specs=pl.BlockSpec((EBLK, D), lambda r, i: (r * N_TBL_BLKS + i, 0)),
        out_shape=jax.ShapeDtypeStruct((NREL * ACC_ROWS, D), jnp.float32),
        compiler_params=pltpu.CompilerParams(
            dimension_semantics=("parallel", "parallel")),
    )(e_pad, weight)


BM = 256
BK = 2048


def _mm_norm_body(a_ref, b_ref, res_ref, out_ref):
    k = pl.program_id(1)
    nk = pl.num_programs(1)

    @pl.when(k == 0)
    def _():
        out_ref[...] = jnp.zeros_like(out_ref)

    @pl.when(k < nk - 1)
    def _():
        out_ref[...] += jnp.dot(a_ref[...], b_ref[...],
                                preferred_element_type=jnp.float32)

    @pl.when(k == nk - 1)
    def _():
        a = a_ref[...]
        col = lax.broadcasted_iota(jnp.int32, a.shape, 1) + k * BK
        a = jnp.where(col < N_ENT, a, 0.0)
        acc = out_ref[...] + jnp.dot(a, b_ref[...],
                                     preferred_element_type=jnp.float32)
        n = jnp.sqrt(jnp.sum(acc * acc, axis=1, keepdims=True))
        out_ref[...] = res_ref[...] + acc / jnp.maximum(n, 1e-12)


def _mm_norm_res(interact_mat, e_pad, usr_res):
    m, kdim = interact_mat.shape
    nk = -(-kdim // BK)
    return pl.pallas_call(
        _mm_norm_body,
        grid=(m // BM, nk),
        in_specs=[pl.BlockSpec((BM, BK), lambda m_, k_: (m_, k_)),
                  pl.BlockSpec((BK, D), lambda m_, k_: (k_, 0)),
                  pl.BlockSpec((BM, D), lambda m_, k_: (m_, 0))],
        out_specs=pl.BlockSpec((BM, D), lambda m_, k_: (m_, 0)),
        out_shape=jax.ShapeDtypeStruct((m, D), jnp.float32),
        compiler_params=pltpu.CompilerParams(
            dimension_semantics=("parallel", "arbitrary")),
    )(interact_mat, e_pad, usr_res)


RBLK = 640


def _combine_body(sums_ref, cnts_ref, res_ref, res_out_ref, e_out_ref):
    s = sums_ref[0] + sums_ref[1]
    c = cnts_ref[0, :, 0] + cnts_ref[1, :, 0]
    e = s / jnp.maximum(c, 1.0)[:, None]
    n = jnp.sqrt(jnp.sum(e * e, axis=1, keepdims=True))
    en = e / jnp.maximum(n, 1e-12)
    res_out_ref[...] = res_ref[...] + en
    e_out_ref[...] = en


def _combine(sums, cnts, ent_res):
    return pl.pallas_call(
        _combine_body,
        grid=(ACC_ROWS // RBLK,),
        in_specs=[pl.BlockSpec((NC, RBLK, D), lambda i: (0, i, 0)),
                  pl.BlockSpec((NC, RBLK, CNT_W), lambda i: (0, i, 0)),
                  pl.BlockSpec((RBLK, D), lambda i: (i, 0))],
        out_specs=[pl.BlockSpec((RBLK, D), lambda i: (i, 0)),
                   pl.BlockSpec((RBLK, D), lambda i: (i, 0))],
        out_shape=[jax.ShapeDtypeStruct((N_ENT, D), jnp.float32),
                   jax.ShapeDtypeStruct((ACC_ROWS, D), jnp.float32)],
        compiler_params=pltpu.CompilerParams(
            dimension_semantics=("parallel",)),
    )(sums, cnts, ent_res)


def kernel(user_emb, entity_emb, edge_index, edge_type, interact_mat, weight):
    head = edge_index[0].astype(jnp.int32)
    tail = edge_index[1].astype(jnp.int32)
    rel_idx = jnp.mod(edge_type - 1, weight.shape[0]).astype(jnp.int32)
    gidx = rel_idx * ACC_ROWS + tail
    gidx2d = jnp.pad(gidx, (0, PAD_EDGES - N_EDGES)).reshape(PAD_CHUNKS, CHUNK)
    head2d = jnp.pad(head, (0, PAD_EDGES - N_EDGES),
                     constant_values=N_ENT).reshape(PAD_CHUNKS, CHUNK)

    e_pad = jnp.pad(entity_emb, ((0, ACC_ROWS - N_ENT), (0, 0)))

    cnts = _sc_cnt(head2d)
    ent_res = entity_emb
    usr_res = user_emb
    for _hop in range(N_HOPS):
        table = _scale_tbl(e_pad, weight)
        sums = _sc_agg(table, gidx2d, head2d)
        usr_res = _mm_norm_res(interact_mat, e_pad, usr_res)
        ent_res, e_pad = _combine(sums, cnts, ent_res)
    return ent_res, usr_res


# speedup vs baseline: 2.4815x; 2.4815x over previous
import jax
import jax.numpy as jnp
from jax import lax
from jax.experimental import pallas as pl
from jax.experimental.pallas import tpu as pltpu
from jax.experimental.pallas import tpu_sc as plsc

N_ENT = 10000
N_USERS = 2048
N_EDGES = 320000
D = 128
NREL = 11
N_HOPS = 2

NC, NS = 2, 16
NW = NC * NS
CHUNK = 128
SUB = 8
SUPER = SUB * CHUNK
N_SUPERS = -(-N_EDGES // SUPER)
SUPERS_PER_TILE = -(-N_SUPERS // NW)
PAD_CHUNKS = N_SUPERS * SUB
PAD_EDGES = PAD_CHUNKS * CHUNK
ROWS_PER_TILE = 640
ACC_ROWS = ROWS_PER_TILE * NS
CNT_W = 16


def _zero_rows(rows_v):
    zeros16 = jnp.zeros((16,), jnp.float32)

    def fill_zero(i, carry):
        for dcol in range(D // 16):
            rows_v[i, pl.ds(dcol * 16, 16)] = zeros16
        return carry

    lax.fori_loop(0, CHUNK, fill_zero, 0)


def _agg_body(table, gidx2d, head2d, sums_out,
              acc, gidx_v, hidx_v, zidx_v, rows0, rows1, sem0, sem1):
    cid = lax.axis_index("c")
    sid = lax.axis_index("s")
    wid = cid * NS + sid

    iota16 = lax.iota(jnp.int32, 16)

    def fill_zidx(base):
        for jj in range(CHUNK // 16):
            zidx_v[pl.ds(jj * 16, 16)] = base + jj * 16 + iota16

    _zero_rows(rows0)
    for j in range(ROWS_PER_TILE // CHUNK):
        base = sid * ROWS_PER_TILE + j * CHUNK
        fill_zidx(base)
        pltpu.sync_copy(rows0, acc.at[zidx_v])

    plsc.subcore_barrier()

    rows = (rows0, rows1)
    sems = (sem0, sem1)

    def super_body(i, carry):
        s = wid + i * NW

        @pl.when(s < N_SUPERS)
        def _():
            pltpu.sync_copy(gidx2d.at[pl.ds(s * SUB, SUB)], gidx_v)
            pltpu.sync_copy(head2d.at[pl.ds(s * SUB, SUB)], hidx_v)
            descs = [None] * SUB
            descs[0] = pltpu.async_copy(table.at[gidx_v.at[0]], rows[0], sems[0])
            for j in range(SUB):
                descs[j].wait()
                if j + 1 < SUB:
                    descs[j + 1] = pltpu.async_copy(
                        table.at[gidx_v.at[j + 1]], rows[(j + 1) % 2],
                        sems[(j + 1) % 2])
                pltpu.sync_copy(rows[j % 2], acc.at[hidx_v.at[j]], add=True)

        return carry

    lax.fori_loop(0, SUPERS_PER_TILE, super_body, 0)

    plsc.subcore_barrier()

    for j in range(ROWS_PER_TILE // CHUNK):
        base = sid * ROWS_PER_TILE + j * CHUNK
        fill_zidx(base)
        pltpu.sync_copy(acc.at[zidx_v], rows0)
        pltpu.sync_copy(rows0, sums_out.at[cid, pl.ds(base, CHUNK)])


def _sc_agg(table, gidx2d, head2d):
    fn = pl.kernel(
        _agg_body,
        out_type=jax.ShapeDtypeStruct((NC, ACC_ROWS, D), jnp.float32),
        mesh=plsc.VectorSubcoreMesh(core_axis_name="c", subcore_axis_name="s",
                                    num_cores=NC, num_subcores=NS),
        scratch_types=[
            pltpu.VMEM_SHARED((ACC_ROWS, D), jnp.float32),
            pltpu.VMEM((SUB, CHUNK), jnp.int32),
            pltpu.VMEM((SUB, CHUNK), jnp.int32),
            pltpu.VMEM((CHUNK,), jnp.int32),
            pltpu.VMEM((CHUNK, D), jnp.float32),
            pltpu.VMEM((CHUNK, D), jnp.float32),
            pltpu.SemaphoreType.DMA,
            pltpu.SemaphoreType.DMA,
        ],
    )
    return fn(table, gidx2d, head2d)


def _cnt_body(head2d, cnts_out, cnt_acc, hidx_v, zidx_v, ones_v):
    cid = lax.axis_index("c")
    sid = lax.axis_index("s")
    wid = cid * NS + sid

    zeros16 = jnp.zeros((16,), jnp.float32)
    ones16 = jnp.ones((16,), jnp.float32)
    iota16 = lax.iota(jnp.int32, 16)

    def fill_zidx(base):
        for jj in range(CHUNK // 16):
            zidx_v[pl.ds(jj * 16, 16)] = base + jj * 16 + iota16

    def fill_ones(val):
        def body(i, carry):
            ones_v[i, :] = val
            return carry

        lax.fori_loop(0, CHUNK, body, 0)

    fill_ones(zeros16)
    for j in range(ROWS_PER_TILE // CHUNK):
        base = sid * ROWS_PER_TILE + j * CHUNK
        fill_zidx(base)
        pltpu.sync_copy(ones_v, cnt_acc.at[zidx_v])
    fill_ones(ones16)

    plsc.subcore_barrier()

    def super_body(i, carry):
        s = wid + i * NW

        @pl.when(s < N_SUPERS)
        def _():
            pltpu.sync_copy(head2d.at[pl.ds(s * SUB, SUB)], hidx_v)
            for j in range(SUB):
                pltpu.sync_copy(ones_v, cnt_acc.at[hidx_v.at[j]], add=True)

        return carry

    lax.fori_loop(0, SUPERS_PER_TILE, super_body, 0)

    plsc.subcore_barrier()

    for j in range(ROWS_PER_TILE // CHUNK):
        base = sid * ROWS_PER_TILE + j * CHUNK
        fill_zidx(base)
        pltpu.sync_copy(cnt_acc.at[zidx_v], ones_v)
        pltpu.sync_copy(ones_v, cnts_out.at[cid, pl.ds(base, CHUNK)])


def _sc_cnt(head2d):
    fn = pl.kernel(
        _cnt_body,
        out_type=jax.ShapeDtypeStruct((NC, ACC_ROWS, CNT_W), jnp.float32),
        mesh=plsc.VectorSubcoreMesh(core_axis_name="c", subcore_axis_name="s",
                                    num_cores=NC, num_subcores=NS),
        scratch_types=[
            pltpu.VMEM_SHARED((ACC_ROWS, CNT_W), jnp.float32),
            pltpu.VMEM((SUB, CHUNK), jnp.int32),
            pltpu.VMEM((CHUNK,), jnp.int32),
            pltpu.VMEM((CHUNK, CNT_W), jnp.float32),
        ],
    )
    return fn(head2d)


EBLK = 2048
N_TBL_BLKS = ACC_ROWS // EBLK


def _scale_tbl_body(e_ref, w_ref, out_ref):
    r = pl.program_id(0)
    out_ref[...] = e_ref[...] * w_ref[pl.ds(r, 1), :]


def _scale_tbl(e_pad, weight):
    return pl.pallas_call(
        _scale_tbl_body,
        grid=(NREL, N_TBL_BLKS),
        in_specs=[pl.BlockSpec((EBLK, D), lambda r, i: (i, 0)),
                  pl.BlockSpec((NREL, D), lambda r, i: (0, 0))],
        out_specs=pl.BlockSpec((EBLK, D), lambda r, i: (r * N_TBL_BLKS + i, 0)),
        out_shape=jax.ShapeDtypeStruct((NREL * ACC_ROWS, D), jnp.float32),
        compiler_params=pltpu.CompilerParams(
            dimension_semantics=("parallel", "parallel")),
    )(e_pad, weight)


BM = 256
BK = 2048


def _mm_norm_body(a_ref, b_ref, res_ref, out_ref):
    k = pl.program_id(1)
    nk = pl.num_programs(1)

    @pl.when(k == 0)
    def _():
        out_ref[...] = jnp.zeros_like(out_ref)

    @pl.when(k < nk - 1)
    def _():
        out_ref[...] += jnp.dot(a_ref[...], b_ref[...],
                                preferred_element_type=jnp.float32)

    @pl.when(k == nk - 1)
    def _():
        a = a_ref[...]
        col = lax.broadcasted_iota(jnp.int32, a.shape, 1) + k * BK
        a = jnp.where(col < N_ENT, a, 0.0)
        acc = out_ref[...] + jnp.dot(a, b_ref[...],
                                     preferred_element_type=jnp.float32)
        n = jnp.sqrt(jnp.sum(acc * acc, axis=1, keepdims=True))
        out_ref[...] = res_ref[...] + acc / jnp.maximum(n, 1e-12)


def _mm_norm_res(interact_mat, e_pad, usr_res):
    m, kdim = interact_mat.shape
    nk = -(-kdim // BK)
    return pl.pallas_call(
        _mm_norm_body,
        grid=(m // BM, nk),
        in_specs=[pl.BlockSpec((BM, BK), lambda m_, k_: (m_, k_)),
                  pl.BlockSpec((BK, D), lambda m_, k_: (k_, 0)),
                  pl.BlockSpec((BM, D), lambda m_, k_: (m_, 0))],
        out_specs=pl.BlockSpec((BM, D), lambda m_, k_: (m_, 0)),
        out_shape=jax.ShapeDtypeStruct((m, D), jnp.float32),
        compiler_params=pltpu.CompilerParams(
            dimension_semantics=("parallel", "arbitrary")),
    )(interact_mat, e_pad, usr_res)


RBLK = 640


def _combine_body(sums_ref, cnts_ref, res_ref, res_out_ref, e_out_ref):
    s = sums_ref[0] + sums_ref[1]
    c = cnts_ref[0, :, 0] + cnts_ref[1, :, 0]
    e = s / jnp.maximum(c, 1.0)[:, None]
    n = jnp.sqrt(jnp.sum(e * e, axis=1, keepdims=True))
    en = e / jnp.maximum(n, 1e-12)
    res_out_ref[...] = res_ref[...] + en
    e_out_ref[...] = en


def _combine(sums, cnts, ent_res):
    return pl.pallas_call(
        _combine_body,
        grid=(ACC_ROWS // RBLK,),
        in_specs=[pl.BlockSpec((NC, RBLK, D), lambda i: (0, i, 0)),
                  pl.BlockSpec((NC, RBLK, CNT_W), lambda i: (0, i, 0)),
                  pl.BlockSpec((RBLK, D), lambda i: (i, 0))],
        out_specs=[pl.BlockSpec((RBLK, D), lambda i: (i, 0)),
                   pl.BlockSpec((RBLK, D), lambda i: (i, 0))],
        out_shape=[jax.ShapeDtypeStruct((N_ENT, D), jnp.float32),
                   jax.ShapeDtypeStruct((ACC_ROWS, D), jnp.float32)],
        compiler_params=pltpu.CompilerParams(
            dimension_semantics=("parallel",)),
    )(sums, cnts, ent_res)


def kernel(user_emb, entity_emb, edge_index, edge_type, interact_mat, weight):
    head = edge_index[0].astype(jnp.int32)
    tail = edge_index[1].astype(jnp.int32)
    rel_idx = jnp.mod(edge_type - 1, weight.shape[0]).astype(jnp.int32)
    gidx = rel_idx * ACC_ROWS + tail
    gidx2d = jnp.pad(gidx, (0, PAD_EDGES - N_EDGES)).reshape(PAD_CHUNKS, CHUNK)
    head2d = jnp.pad(head, (0, PAD_EDGES - N_EDGES),
                     constant_values=N_ENT).reshape(PAD_CHUNKS, CHUNK)

    e_pad = jnp.pad(entity_emb, ((0, ACC_ROWS - N_ENT), (0, 0)))

    cnts = _sc_cnt(head2d)
    ent_res = entity_emb
    usr_res = user_emb
    for _hop in range(N_HOPS):
        table = _scale_tbl(e_pad, weight)
        sums = _sc_agg(table, gidx2d, head2d)
        usr_res = _mm_norm_res(interact_mat, e_pad, usr_res)
        ent_res, e_pad = _combine(sums, cnts, ent_res)
    return ent_res, usr_res


# speedup vs baseline: 2.5439x; 1.0251x over previous
import jax
import jax.numpy as jnp
from jax import lax
from jax.experimental import pallas as pl
from jax.experimental.pallas import tpu as pltpu
from jax.experimental.pallas import tpu_sc as plsc

N_ENT = 10000
N_USERS = 2048
N_EDGES = 320000
D = 128
NREL = 11
N_HOPS = 2

NC, NS = 2, 16
NW = NC * NS
CHUNK = 128
SUB = 4
SUPER = SUB * CHUNK
N_SUPERS = N_EDGES // SUPER
SUPERS_PER_TILE = -(-N_SUPERS // NW)
ROWS_PER_TILE = 640
ACC_ROWS = ROWS_PER_TILE * NS
CNT_W = 16


def _zero_rows(rows_v):
    zeros16 = jnp.zeros((16,), jnp.float32)

    def fill_zero(i, carry):
        for dcol in range(D // 16):
            rows_v[i, pl.ds(dcol * 16, 16)] = zeros16
        return carry

    lax.fori_loop(0, CHUNK, fill_zero, 0)


def _agg_body(table, gidx2d, head2d, sums_out,
              acc, gidx_v, hidx_v, zidx_v, rows0, rows1, sem0, sem1):
    cid = lax.axis_index("c")
    sid = lax.axis_index("s")
    wid = cid * NS + sid

    iota16 = lax.iota(jnp.int32, 16)

    def fill_zidx(base):
        for jj in range(CHUNK // 16):
            zidx_v[pl.ds(jj * 16, 16)] = base + jj * 16 + iota16

    _zero_rows(rows0)
    for j in range(ROWS_PER_TILE // CHUNK):
        base = sid * ROWS_PER_TILE + j * CHUNK
        fill_zidx(base)
        pltpu.sync_copy(rows0, acc.at[zidx_v])

    plsc.subcore_barrier()

    rows = (rows0, rows1)
    sems = (sem0, sem1)

    def super_body(i, carry):
        s = wid + i * NW

        @pl.when(s < N_SUPERS)
        def _():
            pltpu.sync_copy(gidx2d.at[pl.ds(s * SUB, SUB)], gidx_v)
            pltpu.sync_copy(head2d.at[pl.ds(s * SUB, SUB)], hidx_v)
            descs = [None] * SUB
            descs[0] = pltpu.async_copy(table.at[gidx_v.at[0]], rows[0], sems[0])
            for j in range(SUB):
                descs[j].wait()
                if j + 1 < SUB:
                    descs[j + 1] = pltpu.async_copy(
                        table.at[gidx_v.at[j + 1]], rows[(j + 1) % 2],
                        sems[(j + 1) % 2])
                pltpu.sync_copy(rows[j % 2], acc.at[hidx_v.at[j]], add=True)

        return carry

    lax.fori_loop(0, SUPERS_PER_TILE, super_body, 0)

    plsc.subcore_barrier()

    for j in range(ROWS_PER_TILE // CHUNK):
        base = sid * ROWS_PER_TILE + j * CHUNK
        fill_zidx(base)
        pltpu.sync_copy(acc.at[zidx_v], rows0)
        pltpu.sync_copy(rows0, sums_out.at[cid, pl.ds(base, CHUNK)])


def _sc_agg(table, gidx2d, head2d):
    fn = pl.kernel(
        _agg_body,
        out_type=jax.ShapeDtypeStruct((NC, ACC_ROWS, D), jnp.float32),
        mesh=plsc.VectorSubcoreMesh(core_axis_name="c", subcore_axis_name="s",
                                    num_cores=NC, num_subcores=NS),
        scratch_types=[
            pltpu.VMEM_SHARED((ACC_ROWS, D), jnp.float32),
            pltpu.VMEM((SUB, CHUNK), jnp.int32),
            pltpu.VMEM((SUB, CHUNK), jnp.int32),
            pltpu.VMEM((CHUNK,), jnp.int32),
            pltpu.VMEM((CHUNK, D), jnp.float32),
            pltpu.VMEM((CHUNK, D), jnp.float32),
            pltpu.SemaphoreType.DMA,
            pltpu.SemaphoreType.DMA,
        ],
    )
    return fn(table, gidx2d, head2d)


def _cnt_body(head2d, cnts_out, cnt_acc, hidx_v, zidx_v, ones_v):
    cid = lax.axis_index("c")
    sid = lax.axis_index("s")
    wid = cid * NS + sid

    zeros16 = jnp.zeros((16,), jnp.float32)
    ones16 = jnp.ones((16,), jnp.float32)
    iota16 = lax.iota(jnp.int32, 16)

    def fill_zidx(base):
        for jj in range(CHUNK // 16):
            zidx_v[pl.ds(jj * 16, 16)] = base + jj * 16 + iota16

    def fill_ones(val):
        def body(i, carry):
            ones_v[i, :] = val
            return carry

        lax.fori_loop(0, CHUNK, body, 0)

    fill_ones(zeros16)
    for j in range(ROWS_PER_TILE // CHUNK):
        base = sid * ROWS_PER_TILE + j * CHUNK
        fill_zidx(base)
        pltpu.sync_copy(ones_v, cnt_acc.at[zidx_v])
    fill_ones(ones16)

    plsc.subcore_barrier()

    def super_body(i, carry):
        s = wid + i * NW

        @pl.when(s < N_SUPERS)
        def _():
            pltpu.sync_copy(head2d.at[pl.ds(s * SUB, SUB)], hidx_v)
            for j in range(SUB):
                pltpu.sync_copy(ones_v, cnt_acc.at[hidx_v.at[j]], add=True)

        return carry

    lax.fori_loop(0, SUPERS_PER_TILE, super_body, 0)

    plsc.subcore_barrier()

    for j in range(ROWS_PER_TILE // CHUNK):
        base = sid * ROWS_PER_TILE + j * CHUNK
        fill_zidx(base)
        pltpu.sync_copy(cnt_acc.at[zidx_v], ones_v)
        pltpu.sync_copy(ones_v, cnts_out.at[cid, pl.ds(base, CHUNK)])


def _sc_cnt(head2d):
    fn = pl.kernel(
        _cnt_body,
        out_type=jax.ShapeDtypeStruct((NC, ACC_ROWS, CNT_W), jnp.float32),
        mesh=plsc.VectorSubcoreMesh(core_axis_name="c", subcore_axis_name="s",
                                    num_cores=NC, num_subcores=NS),
        scratch_types=[
            pltpu.VMEM_SHARED((ACC_ROWS, CNT_W), jnp.float32),
            pltpu.VMEM((SUB, CHUNK), jnp.int32),
            pltpu.VMEM((CHUNK,), jnp.int32),
            pltpu.VMEM((CHUNK, CNT_W), jnp.float32),
        ],
    )
    return fn(head2d)


EBLK = 2048
N_TBL_BLKS = ACC_ROWS // EBLK


def _scale_tbl_body(e_ref, w_ref, out_ref):
    r = pl.program_id(0)
    out_ref[...] = e_ref[...] * w_ref[pl.ds(r, 1), :]


def _scale_tbl(e_pad, weight):
    return pl.pallas_call(
        _scale_tbl_body,
        grid=(NREL, N_TBL_BLKS),
        in_specs=[pl.BlockSpec((EBLK, D), lambda r, i: (i, 0)),
                  pl.BlockSpec((NREL, D), lambda r, i: (0, 0))],
        out_specs=pl.BlockSpec((EBLK, D), lambda r, i: (r * N_TBL_BLKS + i, 0)),
        out_shape=jax.ShapeDtypeStruct((NREL * ACC_ROWS, D), jnp.float32),
        compiler_params=pltpu.CompilerParams(
            dimension_semantics=("parallel", "parallel")),
    )(e_pad, weight)


BM = 256
BK = 2048


def _mm_norm_body(a_ref, b_ref, res_ref, out_ref):
    k = pl.program_id(1)
    nk = pl.num_programs(1)

    @pl.when(k == 0)
    def _():
        out_ref[...] = jnp.zeros_like(out_ref)

    @pl.when(k < nk - 1)
    def _():
        out_ref[...] += jnp.dot(a_ref[...], b_ref[...],
                                preferred_element_type=jnp.float32)

    @pl.when(k == nk - 1)
    def _():
        a = a_ref[...]
        col = lax.broadcasted_iota(jnp.int32, a.shape, 1) + k * BK
        a = jnp.where(col < N_ENT, a, 0.0)
        acc = out_ref[...] + jnp.dot(a, b_ref[...],
                                     preferred_element_type=jnp.float32)
        n = jnp.sqrt(jnp.sum(acc * acc, axis=1, keepdims=True))
        out_ref[...] = res_ref[...] + acc / jnp.maximum(n, 1e-12)


def _mm_norm_res(interact_mat, e_pad, usr_res):
    m, kdim = interact_mat.shape
    nk = -(-kdim // BK)
    return pl.pallas_call(
        _mm_norm_body,
        grid=(m // BM, nk),
        in_specs=[pl.BlockSpec((BM, BK), lambda m_, k_: (m_, k_)),
                  pl.BlockSpec((BK, D), lambda m_, k_: (k_, 0)),
                  pl.BlockSpec((BM, D), lambda m_, k_: (m_, 0))],
        out_specs=pl.BlockSpec((BM, D), lambda m_, k_: (m_, 0)),
        out_shape=jax.ShapeDtypeStruct((m, D), jnp.float32),
        compiler_params=pltpu.CompilerParams(
            dimension_semantics=("parallel", "arbitrary")),
    )(interact_mat, e_pad, usr_res)


RBLK = 640


def _combine_body(sums_ref, cnts_ref, res_ref, res_out_ref, e_out_ref):
    s = sums_ref[0] + sums_ref[1]
    c = cnts_ref[0, :, 0] + cnts_ref[1, :, 0]
    e = s / jnp.maximum(c, 1.0)[:, None]
    n = jnp.sqrt(jnp.sum(e * e, axis=1, keepdims=True))
    en = e / jnp.maximum(n, 1e-12)
    res_out_ref[...] = res_ref[...] + en
    e_out_ref[...] = en


def _combine(sums, cnts, ent_res):
    return pl.pallas_call(
        _combine_body,
        grid=(ACC_ROWS // RBLK,),
        in_specs=[pl.BlockSpec((NC, RBLK, D), lambda i: (0, i, 0)),
                  pl.BlockSpec((NC, RBLK, CNT_W), lambda i: (0, i, 0)),
                  pl.BlockSpec((RBLK, D), lambda i: (i, 0))],
        out_specs=[pl.BlockSpec((RBLK, D), lambda i: (i, 0)),
                   pl.BlockSpec((RBLK, D), lambda i: (i, 0))],
        out_shape=[jax.ShapeDtypeStruct((N_ENT, D), jnp.float32),
                   jax.ShapeDtypeStruct((ACC_ROWS, D), jnp.float32)],
        compiler_params=pltpu.CompilerParams(
            dimension_semantics=("parallel",)),
    )(sums, cnts, ent_res)


def kernel(user_emb, entity_emb, edge_index, edge_type, interact_mat, weight):
    head = edge_index[0].astype(jnp.int32)
    tail = edge_index[1].astype(jnp.int32)
    rel_idx = jnp.mod(edge_type - 1, weight.shape[0]).astype(jnp.int32)
    gidx2d = (rel_idx * ACC_ROWS + tail).reshape(N_EDGES // CHUNK, CHUNK)
    head2d = head.reshape(N_EDGES // CHUNK, CHUNK)

    e_pad = jnp.pad(entity_emb, ((0, ACC_ROWS - N_ENT), (0, 0)))

    cnts = _sc_cnt(head2d)
    ent_res = entity_emb
    usr_res = user_emb
    for _hop in range(N_HOPS):
        table = _scale_tbl(e_pad, weight)
        sums = _sc_agg(table, gidx2d, head2d)
        usr_res = _mm_norm_res(interact_mat, e_pad, usr_res)
        ent_res, e_pad = _combine(sums, cnts, ent_res)
    return ent_res, usr_res
